# Initial kernel scaffold; baseline (speedup 1.0000x reference)
import jax
import jax.numpy as jnp
from jax.experimental import pallas as pl


def kernel(adj_list, h, enc_W, enc_b, sage_Wl, sage_bl, sage_Wr, Q_W, Q_b, K_W, K_b, V_W, V_b, beta, P_W, P_b):
    raise NotImplementedError("write your pallas kernel here")


# speedup vs baseline: 4.4439x; 4.4439x over previous
import functools

import jax
import jax.numpy as jnp
from jax import lax
from jax.experimental import pallas as pl
from jax.experimental.pallas import tpu as pltpu
from jax.experimental.pallas import tpu_sc as plsc

N = 10000
E = 320000
H = 2
HID = 128
OUT = 64

BN = 1024
NB = 10
NPAD = 10240

NC = 2
NS = 16
CH = 128
NCHUNK = E // CH
BASE_CH = NCHUNK // NS
EXTRA_CH = NCHUNK % NS
RPS = NPAD // NS
ZR = 128
DSTRIPE = NPAD // NS


def _encode_body(h_ref, w_ref, b_ref, o_ref):
    o_ref[0] = jnp.dot(h_ref[...], w_ref[0],
                       preferred_element_type=jnp.float32) + b_ref[0]


def _encode(h, enc_W, enc_b3):
    return pl.pallas_call(
        _encode_body,
        grid=(H, NB),
        in_specs=[
            pl.BlockSpec((BN, HID), lambda i, j: (j, 0)),
            pl.BlockSpec((1, HID, HID), lambda i, j: (i, 0, 0)),
            pl.BlockSpec((1, 1, HID), lambda i, j: (i, 0, 0)),
        ],
        out_specs=pl.BlockSpec((1, BN, HID), lambda i, j: (i, j, 0)),
        out_shape=jax.ShapeDtypeStruct((H, N, HID), jnp.float32),
    )(h, enc_W, enc_b3)


def _sc_agg_body(compute_deg, src_hbm, dst_hbm, xflat_hbm, agg_out, deg_out,
                 idx_s, idx_d, rows, ones_v, zrow, zdeg, agg_sh, deg_sh, sem):
    cid = lax.axis_index("c")
    sid = lax.axis_index("s")

    zk = jnp.zeros((16,), jnp.float32)

    def zero_zrow(r, _):
        for c in range(HID // 16):
            zrow[r, pl.ds(c * 16, 16)] = zk
        return 0

    lax.fori_loop(0, ZR, zero_zrow, 0)

    def fill_vec(vec, val):
        k = jnp.full((16,), val, jnp.float32)

        def body(r, _):
            vec[pl.ds(r * 16, 16)] = k
            return 0

        lax.fori_loop(0, vec.shape[0] // 16, body, 0)

    fill_vec(ones_v, 1.0)
    fill_vec(zdeg, 0.0)

    for k in range(RPS // ZR):
        pltpu.sync_copy(zrow, agg_sh.at[pl.ds(sid * RPS + k * ZR, ZR)])
    if compute_deg:
        pltpu.sync_copy(zdeg, deg_sh.at[pl.ds(sid * DSTRIPE, DSTRIPE)])
    plsc.subcore_barrier()

    head_off = cid * N

    def do_chunk(c):
        off = c * CH
        pltpu.sync_copy(src_hbm.at[cid, pl.ds(off, CH)], idx_s)
        pltpu.sync_copy(dst_hbm.at[cid, pl.ds(off, CH)], idx_d)
        for j in range(CH // 16):
            idx_s[pl.ds(j * 16, 16)] = idx_s[pl.ds(j * 16, 16)] + head_off
        pltpu.async_copy(xflat_hbm.at[idx_s], rows, sem).wait()
        pltpu.sync_copy(rows, agg_sh.at[idx_d], add=True)
        if compute_deg:
            pltpu.sync_copy(ones_v, deg_sh.at[idx_d], add=True)

    def loop_body(t, _):
        do_chunk(sid + t * NS)
        return 0

    lax.fori_loop(0, BASE_CH, loop_body, 0)

    @pl.when(sid < EXTRA_CH)
    def _():
        do_chunk(NS * BASE_CH + sid)

    plsc.subcore_barrier()

    pltpu.sync_copy(agg_sh.at[pl.ds(sid * RPS, RPS)],
                    agg_out.at[cid, pl.ds(sid * RPS, RPS)])
    if compute_deg:
        pltpu.sync_copy(deg_sh.at[pl.ds(sid * DSTRIPE, DSTRIPE)],
                        deg_out.at[cid, pl.ds(sid * DSTRIPE, DSTRIPE)])


def _sc_agg(xflat, src, dst, compute_deg):
    mesh = plsc.VectorSubcoreMesh(core_axis_name="c", subcore_axis_name="s")
    out_type = [jax.ShapeDtypeStruct((H, NPAD, HID), jnp.float32),
                jax.ShapeDtypeStruct((H, NPAD), jnp.float32)]
    scratch = [
        pltpu.VMEM((CH,), jnp.int32),
        pltpu.VMEM((CH,), jnp.int32),
        pltpu.VMEM((CH, HID), jnp.float32),
        pltpu.VMEM((CH,), jnp.float32),
        pltpu.VMEM((ZR, HID), jnp.float32),
        pltpu.VMEM((DSTRIPE,), jnp.float32),
        pltpu.VMEM_SHARED((NPAD, HID), jnp.float32),
        pltpu.VMEM_SHARED((NPAD,), jnp.float32),
        pltpu.SemaphoreType.DMA,
    ]
    fn = pl.kernel(
        functools.partial(_sc_agg_body, compute_deg),
        out_type=out_type,
        mesh=mesh,
        scratch_types=scratch,
    )
    return fn(src, dst, xflat)


def _combine_body(agg_ref, deg_ref, x_ref, wl_ref, bl_ref, wr_ref, o_ref):
    d = jnp.maximum(deg_ref[0], 1.0)
    a = agg_ref[0] / d
    o_ref[0] = (jnp.dot(a, wl_ref[0], preferred_element_type=jnp.float32)
                + bl_ref[0]
                + jnp.dot(x_ref[0], wr_ref[0],
                          preferred_element_type=jnp.float32))


def _combine(agg, deg3, x, Wl, bl3, Wr):
    return pl.pallas_call(
        _combine_body,
        grid=(H, NB),
        in_specs=[
            pl.BlockSpec((1, BN, HID), lambda i, j: (i, j, 0)),
            pl.BlockSpec((1, BN, 1), lambda i, j: (i, j, 0)),
            pl.BlockSpec((1, BN, HID), lambda i, j: (i, j, 0)),
            pl.BlockSpec((1, HID, HID), lambda i, j: (i, 0, 0)),
            pl.BlockSpec((1, 1, HID), lambda i, j: (i, 0, 0)),
            pl.BlockSpec((1, HID, HID), lambda i, j: (i, 0, 0)),
        ],
        out_specs=pl.BlockSpec((1, BN, HID), lambda i, j: (i, j, 0)),
        out_shape=jax.ShapeDtypeStruct((H, N, HID), jnp.float32),
    )(agg, deg3, x, Wl, bl3, Wr)


def _final_body(z_ref, qw_ref, qb_ref, kw_ref, kb_ref, vw_ref, vb_ref,
                beta_ref, pw_ref, pb_ref, o_ref):
    z0 = z_ref[0]
    z1 = z_ref[1]
    f32 = jnp.float32
    q0 = jnp.dot(z0, qw_ref[...], preferred_element_type=f32) + qb_ref[0]
    q1 = jnp.dot(z1, qw_ref[...], preferred_element_type=f32) + qb_ref[0]
    k0 = jnp.dot(z0, kw_ref[...], preferred_element_type=f32) + kb_ref[0]
    k1 = jnp.dot(z1, kw_ref[...], preferred_element_type=f32) + kb_ref[0]
    v0 = jnp.dot(z0, vw_ref[...], preferred_element_type=f32) + vb_ref[0]
    v1 = jnp.dot(z1, vw_ref[...], preferred_element_type=f32) + vb_ref[0]

    def soft2(a, b):
        m = jnp.maximum(a, b)
        ea = jnp.exp(a - m)
        eb = jnp.exp(b - m)
        s = ea + eb
        return ea / s, eb / s

    att00 = jnp.sum(q0 * k0, axis=1, keepdims=True)
    att01 = jnp.sum(q0 * k1, axis=1, keepdims=True)
    att10 = jnp.sum(q1 * k0, axis=1, keepdims=True)
    att11 = jnp.sum(q1 * k1, axis=1, keepdims=True)
    a00, a01 = soft2(att00, att01)
    a10, a11 = soft2(att10, att11)
    b = beta_ref[0, 0]
    r0 = b * (a00 * v0 + a01 * v1) + z1
    r1 = b * (a10 * v0 + a11 * v1) + z1
    o_ref[...] = (jnp.dot(r0, pw_ref[0:HID], preferred_element_type=f32)
                  + jnp.dot(r1, pw_ref[HID:2 * HID],
                            preferred_element_type=f32)
                  + pb_ref[0])


def _final(z, Q_W, Q_b2, K_W, K_b2, V_W, V_b2, beta2, P_W, P_b2):
    full = lambda j: (0, 0)
    return pl.pallas_call(
        _final_body,
        grid=(NB,),
        in_specs=[
            pl.BlockSpec((H, BN, HID), lambda j: (0, j, 0)),
            pl.BlockSpec((HID, HID), full),
            pl.BlockSpec((1, HID), full),
            pl.BlockSpec((HID, HID), full),
            pl.BlockSpec((1, HID), full),
            pl.BlockSpec((HID, HID), full),
            pl.BlockSpec((1, HID), full),
            pl.BlockSpec((1, 1), full),
            pl.BlockSpec((H * HID, OUT), full),
            pl.BlockSpec((1, OUT), full),
        ],
        out_specs=pl.BlockSpec((BN, OUT), lambda j: (j, 0)),
        out_shape=jax.ShapeDtypeStruct((N, OUT), jnp.float32),
    )(z, Q_W, Q_b2, K_W, K_b2, V_W, V_b2, beta2, P_W, P_b2)


def kernel(adj_list, h, enc_W, enc_b, sage_Wl, sage_bl, sage_Wr,
           Q_W, Q_b, K_W, K_b, V_W, V_b, beta, P_W, P_b):
    src = adj_list[:, 0]
    dst = adj_list[:, 1]

    x = _encode(h, enc_W, enc_b.reshape(H, 1, HID))

    agg0, deg = _sc_agg(x.reshape(H * N, HID), src, dst, True)
    deg3 = deg.reshape(H, NPAD, 1)
    x = _combine(agg0, deg3, x,
                 sage_Wl[:, 0], sage_bl[:, 0].reshape(H, 1, HID),
                 sage_Wr[:, 0])

    agg1, _ = _sc_agg(x.reshape(H * N, HID), src, dst, False)
    z = _combine(agg1, deg3, x,
                 sage_Wl[:, 1], sage_bl[:, 1].reshape(H, 1, HID),
                 sage_Wr[:, 1])

    return _final(z, Q_W, Q_b.reshape(1, HID), K_W, K_b.reshape(1, HID),
                  V_W, V_b.reshape(1, HID), beta.reshape(1, 1),
                  P_W, P_b.reshape(1, OUT))
